# baseline (speedup 1.0000x reference)
import functools
import math

import jax
import jax.numpy as jnp
from jax import lax
from jax.experimental import pallas as pl
from jax.experimental.pallas import tpu as pltpu
from jax.experimental.pallas import tpu_sc as plsc

_N = 10000
_E = 160000
_H = 128
_NP = 10240
_EP = 163840
_MAXN = 50
_EPS = 1e-5
_NEG = -1e30

_SC_NW = 32
_SC_CH = 128


def _mm(x, w, b, bm):
    m, k = x.shape
    d = w.shape[1]

    def body(x_ref, w_ref, b_ref, o_ref):
        o_ref[...] = (
            jnp.dot(x_ref[...], w_ref[...], preferred_element_type=jnp.float32)
            + b_ref[...]
        )

    return pl.pallas_call(
        body,
        grid=(m // bm,),
        in_specs=[
            pl.BlockSpec((bm, k), lambda i: (i, 0)),
            pl.BlockSpec((k, d), lambda i: (0, 0)),
            pl.BlockSpec((1, d), lambda i: (0, 0)),
        ],
        out_specs=pl.BlockSpec((bm, d), lambda i: (i, 0)),
        out_shape=jax.ShapeDtypeStruct((m, d), jnp.float32),
    )(x, w, b)


def _flash_attention(q, k, v, n_valid, bq=512, bk=512):
    m, d = q.shape
    nj = m // bk
    scale = 1.0 / math.sqrt(d)

    def body(q_ref, k_ref, v_ref, o_ref, m_ref, l_ref, acc_ref):
        j = pl.program_id(1)

        @pl.when(j == 0)
        def _init():
            m_ref[...] = jnp.full_like(m_ref, _NEG)
            l_ref[...] = jnp.zeros_like(l_ref)
            acc_ref[...] = jnp.zeros_like(acc_ref)

        s = lax.dot_general(
            q_ref[...], k_ref[...], (((1,), (1,)), ((), ())),
            preferred_element_type=jnp.float32,
        ) * scale
        cols = j * bk + lax.broadcasted_iota(jnp.int32, (bq, bk), 1)
        s = jnp.where(cols < n_valid, s, _NEG)

        m_prev = m_ref[...]
        m_new = jnp.maximum(m_prev, jnp.max(s, axis=1, keepdims=True))
        alpha = jnp.exp(m_prev - m_new)
        p = jnp.exp(s - m_new[:, :1])
        l_ref[...] = l_ref[...] * alpha + jnp.sum(p, axis=1, keepdims=True)
        acc_ref[...] = acc_ref[...] * alpha + jnp.dot(
            p, v_ref[...], preferred_element_type=jnp.float32
        )
        m_ref[...] = m_new

        @pl.when(j == nj - 1)
        def _fin():
            o_ref[...] = acc_ref[...] / l_ref[...]

    return pl.pallas_call(
        body,
        grid=(m // bq, nj),
        in_specs=[
            pl.BlockSpec((bq, d), lambda i, j: (i, 0)),
            pl.BlockSpec((bk, d), lambda i, j: (j, 0)),
            pl.BlockSpec((bk, d), lambda i, j: (j, 0)),
        ],
        out_specs=pl.BlockSpec((bq, d), lambda i, j: (i, 0)),
        out_shape=jax.ShapeDtypeStruct((m, d), jnp.float32),
        scratch_shapes=[
            pltpu.VMEM((bq, d), jnp.float32),
            pltpu.VMEM((bq, d), jnp.float32),
            pltpu.VMEM((bq, d), jnp.float32),
        ],
        compiler_params=pltpu.CompilerParams(
            dimension_semantics=("parallel", "arbitrary")
        ),
    )(q, k, v)


def _ln_val(h, g, b):
    mu = jnp.mean(h, axis=-1, keepdims=True)
    c = h - mu
    var = jnp.mean(c * c, axis=-1, keepdims=True)
    return c * lax.rsqrt(var + _EPS) * g + b


def _combine(ao, nm0, nm1, x, wo_t, bo, wl_t, wr_t, bout, g, bln, bm):
    m, d = x.shape

    def body(ao_ref, n0_ref, n1_ref, x_ref, wo_ref, wl_ref, wr_ref,
             bo_ref, bout_ref, g_ref, b_ref, o_ref):
        xa = jnp.dot(ao_ref[...], wo_ref[...],
                     preferred_element_type=jnp.float32) + bo_ref[...]
        h = (
            jnp.dot(xa, wl_ref[...], preferred_element_type=jnp.float32)
            + jnp.dot(n0_ref[...] + n1_ref[...], wr_ref[...],
                      preferred_element_type=jnp.float32)
            + bout_ref[...]
            + x_ref[...]
        )
        o_ref[...] = _ln_val(h, g_ref[...], b_ref[...])

    mat = pl.BlockSpec((bm, d), lambda i: (i, 0))
    wspec = pl.BlockSpec((d, d), lambda i: (0, 0))
    vec = pl.BlockSpec((1, d), lambda i: (0, 0))
    return pl.pallas_call(
        body,
        grid=(m // bm,),
        in_specs=[mat, mat, mat, mat, wspec, wspec, wspec, vec, vec, vec, vec],
        out_specs=mat,
        out_shape=jax.ShapeDtypeStruct((m, d), jnp.float32),
    )(ao, nm0, nm1, x, wo_t, wl_t, wr_t, bo, bout, g, bln)


def _head(x, w1_t, b1, g, bln, w2_t, b2, bm):
    m, d = x.shape
    d1 = w1_t.shape[1]
    d2 = w2_t.shape[1]

    def body(x_ref, w1_ref, b1_ref, g_ref, bln_ref, w2_ref, b2_ref, o_ref):
        h = jnp.dot(x_ref[...], w1_ref[...],
                    preferred_element_type=jnp.float32) + b1_ref[...]
        h = _ln_val(h, g_ref[...], bln_ref[...])
        h = h * (1.0 / (1.0 + jnp.exp(-h)))
        o_ref[...] = jnp.dot(h, w2_ref[...],
                             preferred_element_type=jnp.float32) + b2_ref[...]

    return pl.pallas_call(
        body,
        grid=(m // bm,),
        in_specs=[
            pl.BlockSpec((bm, d), lambda i: (i, 0)),
            pl.BlockSpec((d, d1), lambda i: (0, 0)),
            pl.BlockSpec((1, d1), lambda i: (0, 0)),
            pl.BlockSpec((1, d1), lambda i: (0, 0)),
            pl.BlockSpec((1, d1), lambda i: (0, 0)),
            pl.BlockSpec((d1, d2), lambda i: (0, 0)),
            pl.BlockSpec((1, d2), lambda i: (0, 0)),
        ],
        out_specs=pl.BlockSpec((bm, d2), lambda i: (i, 0)),
        out_shape=jax.ShapeDtypeStruct((m, d2), jnp.float32),
    )(x, w1_t, b1, g, bln, w2_t, b2)


def _mean_topo(x, n_valid, w1_t, b1, w2_t, b2, bm):
    m, d = x.shape
    d1 = w1_t.shape[1]
    d2 = w2_t.shape[1]
    nb = m // bm

    def body(x_ref, w1_ref, b1_ref, w2_ref, b2_ref, o_ref, acc_ref):
        i = pl.program_id(0)

        @pl.when(i == 0)
        def _init():
            acc_ref[...] = jnp.zeros_like(acc_ref)

        rows = i * bm + lax.broadcasted_iota(jnp.int32, (bm, d), 0)
        xm = jnp.where(rows < n_valid, x_ref[...], 0.0)
        acc_ref[...] = acc_ref[...] + jnp.sum(xm, axis=0, keepdims=True)

        @pl.when(i == nb - 1)
        def _fin():
            gr = acc_ref[...] * (1.0 / n_valid)
            h = jnp.maximum(
                jnp.dot(gr, w1_ref[...], preferred_element_type=jnp.float32)
                + b1_ref[...], 0.0)
            o_ref[...] = jnp.dot(h, w2_ref[...],
                                 preferred_element_type=jnp.float32) + b2_ref[...]

    return pl.pallas_call(
        body,
        grid=(nb,),
        in_specs=[
            pl.BlockSpec((bm, d), lambda i: (i, 0)),
            pl.BlockSpec((d, d1), lambda i: (0, 0)),
            pl.BlockSpec((1, d1), lambda i: (0, 0)),
            pl.BlockSpec((d1, d2), lambda i: (0, 0)),
            pl.BlockSpec((1, d2), lambda i: (0, 0)),
        ],
        out_specs=pl.BlockSpec((1, d2), lambda i: (0, 0)),
        out_shape=jax.ShapeDtypeStruct((1, d2), jnp.float32),
        scratch_shapes=[pltpu.VMEM((1, d), jnp.float32)],
        compiler_params=pltpu.CompilerParams(
            dimension_semantics=("arbitrary",)
        ),
    )(x, w1_t, b1, w2_t, b2)


@functools.cache
def _sc_mesh():
    return plsc.VectorSubcoreMesh(core_axis_name="c", subcore_axis_name="s")


def _sc_edge_layer(a, b, c, row_idx, col_idx, g, bln, n_rows):
    ep = c.shape[0]
    perw = ep // _SC_NW
    ch = 64
    nch = perw // ch
    rpt = n_rows // 16
    idx_t = pltpu.VMEM((ch,), jnp.int32)
    buf_t = pltpu.VMEM((ch, _H), jnp.float32)
    sem_t = pltpu.SemaphoreType.DMA

    @functools.partial(
        pl.kernel,
        mesh=_sc_mesh(),
        out_type=jax.ShapeDtypeStruct((2, n_rows, _H), jnp.float32),
        scratch_types=[
            (idx_t, idx_t), (idx_t, idx_t),
            (buf_t, buf_t), (buf_t, buf_t), buf_t,
            pltpu.VMEM((_H,), jnp.float32),
            pltpu.VMEM((_H,), jnp.float32),
            pltpu.VMEM_SHARED((n_rows, _H), jnp.float32),
            (sem_t, sem_t), (sem_t, sem_t), sem_t,
            (sem_t, sem_t), (sem_t, sem_t),
        ],
    )
    def k(a_hbm, b_hbm, c_hbm, row_hbm, col_hbm, g_hbm, bln_hbm, out_hbm,
          row_v, col_v, abuf, bbuf, cbuf, gbuf, blnbuf, acc,
          sem_a, sem_b, sem_c, sem_ri, sem_ci):
        cid = lax.axis_index("c")
        sid = lax.axis_index("s")
        wid = sid * 2 + cid

        pltpu.sync_copy(g_hbm, gbuf)
        pltpu.sync_copy(bln_hbm, blnbuf)

        def zb(r, cr):
            for u in range(_H // 16):
                cbuf[r, pl.ds(u * 16, 16)] = jnp.zeros((16,), jnp.float32)
            return cr

        lax.fori_loop(0, ch, zb, 0)
        for t in range(rpt // ch):
            pltpu.sync_copy(cbuf, acc.at[pl.ds(sid * rpt + t * ch, ch)])
        plsc.subcore_barrier()

        def a_stage(ci, bu):
            base = wid * perw + ci * ch
            pltpu.async_copy(row_hbm.at[pl.ds(base, ch)], row_v[bu],
                             sem_ri[bu])
            pltpu.async_copy(col_hbm.at[pl.ds(base, ch)], col_v[bu],
                             sem_ci[bu])

        def b_stage(ci, bu):
            base = wid * perw + ci * ch
            pltpu.make_async_copy(row_hbm.at[pl.ds(base, ch)], row_v[bu],
                                  sem_ri[bu]).wait()
            pltpu.make_async_copy(col_hbm.at[pl.ds(base, ch)], col_v[bu],
                                  sem_ci[bu]).wait()
            pltpu.async_copy(a_hbm.at[row_v[bu]], abuf[bu], sem_a[bu])
            pltpu.async_copy(b_hbm.at[col_v[bu]], bbuf[bu], sem_b[bu])

        def c_stage(ci, bu):
            base = wid * perw + ci * ch
            cpc = pltpu.async_copy(c_hbm.at[pl.ds(base, ch)], cbuf, sem_c)
            pltpu.make_async_copy(a_hbm.at[row_v[bu]], abuf[bu],
                                  sem_a[bu]).wait()
            pltpu.make_async_copy(b_hbm.at[col_v[bu]], bbuf[bu],
                                  sem_b[bu]).wait()
            cpc.wait()
            ab, bb, cb = abuf[bu], bbuf[bu], cbuf

            def one_row(r):
                t = [ab[r, pl.ds(u * 16, 16)] + bb[r, pl.ds(u * 16, 16)]
                     + cb[r, pl.ds(u * 16, 16)] for u in range(_H // 16)]
                s = t[0]
                sq = t[0] * t[0]
                for u in range(1, _H // 16):
                    s = s + t[u]
                    sq = sq + t[u] * t[u]
                lane = lax.iota(jnp.int32, 16)
                dnum = lax.GatherDimensionNumbers(
                    offset_dims=(), collapsed_slice_dims=(0,),
                    start_index_map=(0,))

                def shuf(v, perm):
                    return lax.gather(
                        v, perm[:, None], dnum, slice_sizes=(1,),
                        mode=lax.GatherScatterMode.PROMISE_IN_BOUNDS)

                for kk in (1, 2, 4, 8):
                    perm = lane ^ kk
                    s = s + shuf(s, perm)
                    sq = sq + shuf(sq, perm)
                meanv = s * (1.0 / _H)
                var = sq * (1.0 / _H) - meanv * meanv
                xv = var + _EPS
                xi = lax.bitcast_convert_type(xv, jnp.int32)
                yi = 0x5F3759DF - lax.shift_right_arithmetic(xi, 1)
                y = lax.bitcast_convert_type(yi, jnp.float32)
                y = y * (1.5 - 0.5 * xv * y * y)
                y = y * (1.5 - 0.5 * xv * y * y)
                y = y * (1.5 - 0.5 * xv * y * y)
                for u in range(_H // 16):
                    sl = pl.ds(u * 16, 16)
                    o = (t[u] - meanv) * y * gbuf[sl] + blnbuf[sl]
                    cb[r, sl] = jnp.maximum(o, 0.0)

            def rbody(r4, cr):
                one_row(r4 * 4)
                one_row(r4 * 4 + 1)
                one_row(r4 * 4 + 2)
                one_row(r4 * 4 + 3)
                return cr

            lax.fori_loop(0, ch // 4, rbody, 0)
            pltpu.sync_copy(cb, acc.at[col_v[bu]], add=True)

        a_stage(0, 0)
        a_stage(1, 1)
        b_stage(0, 0)

        def gbody(gi, cr):
            b_stage(2 * gi + 1, 1)
            c_stage(2 * gi, 0)

            @pl.when(gi < nch // 2 - 1)
            def _n0():
                a_stage(2 * gi + 2, 0)
                b_stage(2 * gi + 2, 0)

            c_stage(2 * gi + 1, 1)

            @pl.when(gi < nch // 2 - 1)
            def _n1():
                a_stage(2 * gi + 3, 1)

            return cr

        lax.fori_loop(0, nch // 2, gbody, 0)
        plsc.subcore_barrier()
        pltpu.sync_copy(acc.at[pl.ds(sid * rpt, rpt)],
                        out_hbm.at[cid, pl.ds(sid * rpt, rpt)])

    return k(a, b, c, row_idx, col_idx, g, bln)


def _sc_gather_mean(x, row_idx, col_idx):
    ep = row_idx.shape[0]
    perw = ep // _SC_NW
    nch = perw // _SC_CH

    idx_t = pltpu.VMEM((_SC_CH,), jnp.int32)
    buf_t = pltpu.VMEM((_SC_CH, _H), jnp.float32)
    sem_t = pltpu.SemaphoreType.DMA

    @functools.partial(
        pl.kernel,
        mesh=_sc_mesh(),
        out_type=jax.ShapeDtypeStruct((ep, _H), jnp.float32),
        scratch_types=[
            (idx_t, idx_t), (idx_t, idx_t),
            (buf_t, buf_t), (buf_t, buf_t),
            (sem_t, sem_t), (sem_t, sem_t),
            (sem_t, sem_t), (sem_t, sem_t),
        ],
    )
    def k(x_hbm, row_hbm, col_hbm, out_hbm,
          row_v, col_v, abuf, bbuf, sem_a, sem_b, sem_ri, sem_ci):
        wid = lax.axis_index("s") * 2 + lax.axis_index("c")

        def a_stage(ci, bu):
            base = wid * perw + ci * _SC_CH
            pltpu.async_copy(row_hbm.at[pl.ds(base, _SC_CH)], row_v[bu],
                             sem_ri[bu])
            pltpu.async_copy(col_hbm.at[pl.ds(base, _SC_CH)], col_v[bu],
                             sem_ci[bu])

        def b_stage(ci, bu):
            base = wid * perw + ci * _SC_CH
            pltpu.make_async_copy(row_hbm.at[pl.ds(base, _SC_CH)], row_v[bu],
                                  sem_ri[bu]).wait()
            pltpu.make_async_copy(col_hbm.at[pl.ds(base, _SC_CH)], col_v[bu],
                                  sem_ci[bu]).wait()
            pltpu.async_copy(x_hbm.at[row_v[bu]], abuf[bu], sem_a[bu])
            pltpu.async_copy(x_hbm.at[col_v[bu]], bbuf[bu], sem_b[bu])

        def c_stage(ci, bu):
            base = wid * perw + ci * _SC_CH
            pltpu.make_async_copy(x_hbm.at[row_v[bu]], abuf[bu],
                                  sem_a[bu]).wait()
            pltpu.make_async_copy(x_hbm.at[col_v[bu]], bbuf[bu],
                                  sem_b[bu]).wait()
            ab, bb = abuf[bu], bbuf[bu]

            def rbody(r4, cr):
                for rr in range(4):
                    r = r4 * 4 + rr
                    for u in range(_H // 16):
                        sl = pl.ds(u * 16, 16)
                        ab[r, sl] = (ab[r, sl] + bb[r, sl]) * 0.5
                return cr

            lax.fori_loop(0, _SC_CH // 4, rbody, 0)
            pltpu.sync_copy(ab, out_hbm.at[pl.ds(base, _SC_CH)])

        a_stage(0, 0)
        a_stage(1, 1)
        b_stage(0, 0)

        def gbody(gi, cr):
            b_stage(2 * gi + 1, 1)
            c_stage(2 * gi, 0)

            @pl.when(gi < nch // 2 - 1)
            def _n0():
                a_stage(2 * gi + 2, 0)
                b_stage(2 * gi + 2, 0)

            c_stage(2 * gi + 1, 1)

            @pl.when(gi < nch // 2 - 1)
            def _n1():
                a_stage(2 * gi + 3, 1)

            return cr

        lax.fori_loop(0, nch // 2, gbody, 0)

    return k(x, row_idx, col_idx)


def kernel(node_features, edge_features, edge_index, global_features,
           timestep, params):
    del global_features
    p = params
    f32 = jnp.float32

    x0 = jnp.pad(node_features, ((0, _NP - _N), (0, 0)))
    ef = jnp.pad(edge_features, ((0, _EP - _E), (0, 0)))
    row = edge_index[0]
    col = edge_index[1]
    rowp = jnp.concatenate([row, jnp.zeros((_EP - _E,), jnp.int32)])
    colp = jnp.concatenate(
        [col, jnp.full((_EP - _E,), _NP - 1, jnp.int32)])

    t = timestep.astype(f32)[:, None]
    h1 = t @ p["time_l1"]["w"].T + p["time_l1"]["b"]
    h1 = h1 * jax.nn.sigmoid(h1)
    t_emb = h1 @ p["time_l2"]["w"].T + p["time_l2"]["b"]

    node_b = p["node_proj"]["b"][None, :] + t_emb
    x = _mm(x0, p["node_proj"]["w"].T, node_b, bm=512)

    for lp in p["layers"]:
        wm = lp["mlp_lin"]["w"]
        wr_, wc_, we_ = wm[:, :_H], wm[:, _H:2 * _H], wm[:, 2 * _H:]
        big_w = jnp.concatenate(
            [lp["q"]["w"].T, lp["k"]["w"].T, lp["v"]["w"].T, wr_.T, wc_.T],
            axis=1)
        big_b = jnp.concatenate(
            [lp["q"]["b"], lp["k"]["b"], lp["v"]["b"],
             jnp.zeros((2 * _H,), f32)])[None, :]
        big = _mm(x, big_w, big_b, bm=512)
        q = big[:, :_H]
        kk = big[:, _H:2 * _H]
        vv = big[:, 2 * _H:3 * _H]
        a_rows = big[:, 3 * _H:4 * _H]
        b_rows = big[:, 4 * _H:]

        ao = _flash_attention(q, kk, vv, _N)

        cw = p["edge_proj"]["w"].T @ we_.T
        cb = (p["edge_proj"]["b"] @ we_.T + lp["mlp_lin"]["b"])[None, :]
        c_rows = _mm(ef, cw, cb, bm=1280)

        nm = _sc_edge_layer(a_rows, b_rows, c_rows, rowp, colp,
                            lp["mlp_ln"]["g"], lp["mlp_ln"]["b"],
                            _NP)

        wout = lp["out"]["w"]
        x = _combine(
            ao, nm[0], nm[1], x,
            lp["o"]["w"].T, lp["o"]["b"][None, :],
            wout[:, :_H].T, wout[:, _H:].T, lp["out"]["b"][None, :],
            lp["ln"]["g"][None, :], lp["ln"]["b"][None, :], bm=512)

    node_noise = _head(
        x, p["node_out1"]["w"].T, p["node_out1"]["b"][None, :],
        p["node_out_ln"]["g"][None, :], p["node_out_ln"]["b"][None, :],
        p["node_out2"]["w"].T, p["node_out2"]["b"][None, :], bm=512)[:_N]

    g2 = _sc_gather_mean(x, rowp, colp)
    edge_noise = _head(
        g2, p["edge_out1"]["w"].T, p["edge_out1"]["b"][None, :],
        p["edge_out_ln"]["g"][None, :], p["edge_out_ln"]["b"][None, :],
        p["edge_out2"]["w"].T, p["edge_out2"]["b"][None, :], bm=1280)[:_E]

    topo = _mean_topo(
        x, _N, p["topo1"]["w"].T, p["topo1"]["b"][None, :],
        p["topo2"]["w"].T, p["topo2"]["b"][None, :], bm=512)
    topology_logits = topo.reshape(_MAXN, _MAXN)

    return (node_noise, edge_noise, topology_logits, x[:_N])


# speedup vs baseline: 1.0406x; 1.0406x over previous
import functools
import math

import jax
import jax.numpy as jnp
from jax import lax
from jax.experimental import pallas as pl
from jax.experimental.pallas import tpu as pltpu
from jax.experimental.pallas import tpu_sc as plsc

_N = 10000
_E = 160000
_H = 128
_NP = 10240
_EP = 163840
_MAXN = 50
_EPS = 1e-5
_NEG = -1e30

_SC_NW = 32
_SC_CH = 128


def _mm(x, w, b, bm):
    m, k = x.shape
    d = w.shape[1]

    def body(x_ref, w_ref, b_ref, o_ref):
        o_ref[...] = (
            jnp.dot(x_ref[...], w_ref[...], preferred_element_type=jnp.float32)
            + b_ref[...]
        )

    return pl.pallas_call(
        body,
        grid=(m // bm,),
        in_specs=[
            pl.BlockSpec((bm, k), lambda i: (i, 0)),
            pl.BlockSpec((k, d), lambda i: (0, 0)),
            pl.BlockSpec((1, d), lambda i: (0, 0)),
        ],
        out_specs=pl.BlockSpec((bm, d), lambda i: (i, 0)),
        out_shape=jax.ShapeDtypeStruct((m, d), jnp.float32),
    )(x, w, b)


def _flash_attention(q, k, v, n_valid, bq=512, bk=512):
    m, d = q.shape
    nj = m // bk
    scale = 1.0 / math.sqrt(d)

    def body(q_ref, k_ref, v_ref, o_ref, m_ref, l_ref, acc_ref):
        j = pl.program_id(1)

        @pl.when(j == 0)
        def _init():
            m_ref[...] = jnp.full_like(m_ref, _NEG)
            l_ref[...] = jnp.zeros_like(l_ref)
            acc_ref[...] = jnp.zeros_like(acc_ref)

        s = lax.dot_general(
            q_ref[...], k_ref[...], (((1,), (1,)), ((), ())),
            preferred_element_type=jnp.float32,
        ) * scale
        cols = j * bk + lax.broadcasted_iota(jnp.int32, (bq, bk), 1)
        s = jnp.where(cols < n_valid, s, _NEG)

        m_prev = m_ref[...]
        m_new = jnp.maximum(m_prev, jnp.max(s, axis=1, keepdims=True))
        alpha = jnp.exp(m_prev - m_new)
        p = jnp.exp(s - m_new[:, :1])
        l_ref[...] = l_ref[...] * alpha + jnp.sum(p, axis=1, keepdims=True)
        acc_ref[...] = acc_ref[...] * alpha + jnp.dot(
            p, v_ref[...], preferred_element_type=jnp.float32
        )
        m_ref[...] = m_new

        @pl.when(j == nj - 1)
        def _fin():
            o_ref[...] = acc_ref[...] / l_ref[...]

    return pl.pallas_call(
        body,
        grid=(m // bq, nj),
        in_specs=[
            pl.BlockSpec((bq, d), lambda i, j: (i, 0)),
            pl.BlockSpec((bk, d), lambda i, j: (j, 0)),
            pl.BlockSpec((bk, d), lambda i, j: (j, 0)),
        ],
        out_specs=pl.BlockSpec((bq, d), lambda i, j: (i, 0)),
        out_shape=jax.ShapeDtypeStruct((m, d), jnp.float32),
        scratch_shapes=[
            pltpu.VMEM((bq, d), jnp.float32),
            pltpu.VMEM((bq, d), jnp.float32),
            pltpu.VMEM((bq, d), jnp.float32),
        ],
        compiler_params=pltpu.CompilerParams(
            dimension_semantics=("parallel", "arbitrary")
        ),
    )(q, k, v)


def _ln_val(h, g, b):
    mu = jnp.mean(h, axis=-1, keepdims=True)
    c = h - mu
    var = jnp.mean(c * c, axis=-1, keepdims=True)
    return c * lax.rsqrt(var + _EPS) * g + b


def _combine(ao, nm0, nm1, x, wo_t, bo, wl_t, wr_t, bout, g, bln, bm):
    m, d = x.shape

    def body(ao_ref, n0_ref, n1_ref, x_ref, wo_ref, wl_ref, wr_ref,
             bo_ref, bout_ref, g_ref, b_ref, o_ref):
        xa = jnp.dot(ao_ref[...], wo_ref[...],
                     preferred_element_type=jnp.float32) + bo_ref[...]
        h = (
            jnp.dot(xa, wl_ref[...], preferred_element_type=jnp.float32)
            + jnp.dot(n0_ref[...] + n1_ref[...], wr_ref[...],
                      preferred_element_type=jnp.float32)
            + bout_ref[...]
            + x_ref[...]
        )
        o_ref[...] = _ln_val(h, g_ref[...], b_ref[...])

    mat = pl.BlockSpec((bm, d), lambda i: (i, 0))
    wspec = pl.BlockSpec((d, d), lambda i: (0, 0))
    vec = pl.BlockSpec((1, d), lambda i: (0, 0))
    return pl.pallas_call(
        body,
        grid=(m // bm,),
        in_specs=[mat, mat, mat, mat, wspec, wspec, wspec, vec, vec, vec, vec],
        out_specs=mat,
        out_shape=jax.ShapeDtypeStruct((m, d), jnp.float32),
    )(ao, nm0, nm1, x, wo_t, wl_t, wr_t, bo, bout, g, bln)


def _head(x, w1_t, b1, g, bln, w2_t, b2, bm):
    m, d = x.shape
    d1 = w1_t.shape[1]
    d2 = w2_t.shape[1]

    def body(x_ref, w1_ref, b1_ref, g_ref, bln_ref, w2_ref, b2_ref, o_ref):
        h = jnp.dot(x_ref[...], w1_ref[...],
                    preferred_element_type=jnp.float32) + b1_ref[...]
        h = _ln_val(h, g_ref[...], bln_ref[...])
        h = h * (1.0 / (1.0 + jnp.exp(-h)))
        o_ref[...] = jnp.dot(h, w2_ref[...],
                             preferred_element_type=jnp.float32) + b2_ref[...]

    return pl.pallas_call(
        body,
        grid=(m // bm,),
        in_specs=[
            pl.BlockSpec((bm, d), lambda i: (i, 0)),
            pl.BlockSpec((d, d1), lambda i: (0, 0)),
            pl.BlockSpec((1, d1), lambda i: (0, 0)),
            pl.BlockSpec((1, d1), lambda i: (0, 0)),
            pl.BlockSpec((1, d1), lambda i: (0, 0)),
            pl.BlockSpec((d1, d2), lambda i: (0, 0)),
            pl.BlockSpec((1, d2), lambda i: (0, 0)),
        ],
        out_specs=pl.BlockSpec((bm, d2), lambda i: (i, 0)),
        out_shape=jax.ShapeDtypeStruct((m, d2), jnp.float32),
    )(x, w1_t, b1, g, bln, w2_t, b2)


def _mean_topo(x, n_valid, w1_t, b1, w2_t, b2, bm):
    m, d = x.shape
    d1 = w1_t.shape[1]
    d2 = w2_t.shape[1]
    nb = m // bm

    def body(x_ref, w1_ref, b1_ref, w2_ref, b2_ref, o_ref, acc_ref):
        i = pl.program_id(0)

        @pl.when(i == 0)
        def _init():
            acc_ref[...] = jnp.zeros_like(acc_ref)

        rows = i * bm + lax.broadcasted_iota(jnp.int32, (bm, d), 0)
        xm = jnp.where(rows < n_valid, x_ref[...], 0.0)
        acc_ref[...] = acc_ref[...] + jnp.sum(xm, axis=0, keepdims=True)

        @pl.when(i == nb - 1)
        def _fin():
            gr = acc_ref[...] * (1.0 / n_valid)
            h = jnp.maximum(
                jnp.dot(gr, w1_ref[...], preferred_element_type=jnp.float32)
                + b1_ref[...], 0.0)
            o_ref[...] = jnp.dot(h, w2_ref[...],
                                 preferred_element_type=jnp.float32) + b2_ref[...]

    return pl.pallas_call(
        body,
        grid=(nb,),
        in_specs=[
            pl.BlockSpec((bm, d), lambda i: (i, 0)),
            pl.BlockSpec((d, d1), lambda i: (0, 0)),
            pl.BlockSpec((1, d1), lambda i: (0, 0)),
            pl.BlockSpec((d1, d2), lambda i: (0, 0)),
            pl.BlockSpec((1, d2), lambda i: (0, 0)),
        ],
        out_specs=pl.BlockSpec((1, d2), lambda i: (0, 0)),
        out_shape=jax.ShapeDtypeStruct((1, d2), jnp.float32),
        scratch_shapes=[pltpu.VMEM((1, d), jnp.float32)],
        compiler_params=pltpu.CompilerParams(
            dimension_semantics=("arbitrary",)
        ),
    )(x, w1_t, b1, w2_t, b2)


@functools.cache
def _sc_mesh():
    return plsc.VectorSubcoreMesh(core_axis_name="c", subcore_axis_name="s")


def _sc_edge_layer(a, b, c, row_idx, col_idx, g, bln, n_rows):
    ep = c.shape[0]
    perw = ep // _SC_NW
    ch = 64
    nch = perw // ch
    rpt = n_rows // 16
    idx_t = pltpu.VMEM((ch,), jnp.int32)
    buf_t = pltpu.VMEM((ch, _H), jnp.float32)
    sem_t = pltpu.SemaphoreType.DMA

    @functools.partial(
        pl.kernel,
        mesh=_sc_mesh(),
        out_type=jax.ShapeDtypeStruct((2, n_rows, _H), jnp.float32),
        scratch_types=[
            (idx_t, idx_t), (idx_t, idx_t),
            (buf_t, buf_t), (buf_t, buf_t), buf_t,
            pltpu.VMEM((_H,), jnp.float32),
            pltpu.VMEM((_H,), jnp.float32),
            pltpu.VMEM_SHARED((n_rows, _H), jnp.float32),
            (sem_t, sem_t), (sem_t, sem_t), sem_t,
            (sem_t, sem_t), (sem_t, sem_t),
        ],
    )
    def k(a_hbm, b_hbm, c_hbm, row_hbm, col_hbm, g_hbm, bln_hbm, out_hbm,
          row_v, col_v, abuf, bbuf, cbuf, gbuf, blnbuf, acc,
          sem_a, sem_b, sem_c, sem_ri, sem_ci):
        cid = lax.axis_index("c")
        sid = lax.axis_index("s")
        wid = sid * 2 + cid

        pltpu.sync_copy(g_hbm, gbuf)
        pltpu.sync_copy(bln_hbm, blnbuf)

        def zb(r, cr):
            for u in range(_H // 16):
                cbuf[r, pl.ds(u * 16, 16)] = jnp.zeros((16,), jnp.float32)
            return cr

        lax.fori_loop(0, ch, zb, 0)
        for t in range(rpt // ch):
            pltpu.sync_copy(cbuf, acc.at[pl.ds(sid * rpt + t * ch, ch)])
        plsc.subcore_barrier()

        def a_stage(ci, bu):
            base = wid * perw + ci * ch
            pltpu.async_copy(row_hbm.at[pl.ds(base, ch)], row_v[bu],
                             sem_ri[bu])
            pltpu.async_copy(col_hbm.at[pl.ds(base, ch)], col_v[bu],
                             sem_ci[bu])

        def b_stage(ci, bu):
            base = wid * perw + ci * ch
            pltpu.make_async_copy(row_hbm.at[pl.ds(base, ch)], row_v[bu],
                                  sem_ri[bu]).wait()
            pltpu.make_async_copy(col_hbm.at[pl.ds(base, ch)], col_v[bu],
                                  sem_ci[bu]).wait()
            pltpu.async_copy(a_hbm.at[row_v[bu]], abuf[bu], sem_a[bu])
            pltpu.async_copy(b_hbm.at[col_v[bu]], bbuf[bu], sem_b[bu])

        def c_stage(ci, bu):
            base = wid * perw + ci * ch
            cpc = pltpu.async_copy(c_hbm.at[pl.ds(base, ch)], cbuf, sem_c)
            pltpu.make_async_copy(a_hbm.at[row_v[bu]], abuf[bu],
                                  sem_a[bu]).wait()
            pltpu.make_async_copy(b_hbm.at[col_v[bu]], bbuf[bu],
                                  sem_b[bu]).wait()
            cpc.wait()
            ab, bb, cb = abuf[bu], bbuf[bu], cbuf

            def one_row(r):
                t = [ab[r, pl.ds(u * 16, 16)] + bb[r, pl.ds(u * 16, 16)]
                     + cb[r, pl.ds(u * 16, 16)] for u in range(_H // 16)]
                s = t[0]
                sq = t[0] * t[0]
                for u in range(1, _H // 16):
                    s = s + t[u]
                    sq = sq + t[u] * t[u]
                lane = lax.iota(jnp.int32, 16)
                dnum = lax.GatherDimensionNumbers(
                    offset_dims=(), collapsed_slice_dims=(0,),
                    start_index_map=(0,))

                def shuf(v, perm):
                    return lax.gather(
                        v, perm[:, None], dnum, slice_sizes=(1,),
                        mode=lax.GatherScatterMode.PROMISE_IN_BOUNDS)

                for kk in (1, 2, 4, 8):
                    perm = lane ^ kk
                    s = s + shuf(s, perm)
                    sq = sq + shuf(sq, perm)
                meanv = s * (1.0 / _H)
                var = sq * (1.0 / _H) - meanv * meanv
                xv = var + _EPS
                xi = lax.bitcast_convert_type(xv, jnp.int32)
                yi = 0x5F3759DF - lax.shift_right_arithmetic(xi, 1)
                y = lax.bitcast_convert_type(yi, jnp.float32)
                y = y * (1.5 - 0.5 * xv * y * y)
                y = y * (1.5 - 0.5 * xv * y * y)
                y = y * (1.5 - 0.5 * xv * y * y)
                for u in range(_H // 16):
                    sl = pl.ds(u * 16, 16)
                    o = (t[u] - meanv) * y * gbuf[sl] + blnbuf[sl]
                    cb[r, sl] = jnp.maximum(o, 0.0)

            def rbody(r2, cr):
                one_row(r2 * 2)
                one_row(r2 * 2 + 1)
                return cr

            lax.fori_loop(0, ch // 2, rbody, 0)
            pltpu.sync_copy(cb, acc.at[col_v[bu]], add=True)

        a_stage(0, 0)
        a_stage(1, 1)
        b_stage(0, 0)

        def gbody(gi, cr):
            b_stage(2 * gi + 1, 1)
            c_stage(2 * gi, 0)

            @pl.when(gi < nch // 2 - 1)
            def _n0():
                a_stage(2 * gi + 2, 0)
                b_stage(2 * gi + 2, 0)

            c_stage(2 * gi + 1, 1)

            @pl.when(gi < nch // 2 - 1)
            def _n1():
                a_stage(2 * gi + 3, 1)

            return cr

        lax.fori_loop(0, nch // 2, gbody, 0)
        plsc.subcore_barrier()
        pltpu.sync_copy(acc.at[pl.ds(sid * rpt, rpt)],
                        out_hbm.at[cid, pl.ds(sid * rpt, rpt)])

    return k(a, b, c, row_idx, col_idx, g, bln)


def _sc_gather_mean(x, row_idx, col_idx):
    ep = row_idx.shape[0]
    perw = ep // _SC_NW
    nch = perw // _SC_CH

    idx_t = pltpu.VMEM((_SC_CH,), jnp.int32)
    buf_t = pltpu.VMEM((_SC_CH, _H), jnp.float32)
    sem_t = pltpu.SemaphoreType.DMA

    @functools.partial(
        pl.kernel,
        mesh=_sc_mesh(),
        out_type=jax.ShapeDtypeStruct((ep, _H), jnp.float32),
        scratch_types=[
            (idx_t, idx_t), (idx_t, idx_t),
            (buf_t, buf_t), (buf_t, buf_t),
            (sem_t, sem_t), (sem_t, sem_t),
            (sem_t, sem_t), (sem_t, sem_t),
        ],
    )
    def k(x_hbm, row_hbm, col_hbm, out_hbm,
          row_v, col_v, abuf, bbuf, sem_a, sem_b, sem_ri, sem_ci):
        wid = lax.axis_index("s") * 2 + lax.axis_index("c")

        def a_stage(ci, bu):
            base = wid * perw + ci * _SC_CH
            pltpu.async_copy(row_hbm.at[pl.ds(base, _SC_CH)], row_v[bu],
                             sem_ri[bu])
            pltpu.async_copy(col_hbm.at[pl.ds(base, _SC_CH)], col_v[bu],
                             sem_ci[bu])

        def b_stage(ci, bu):
            base = wid * perw + ci * _SC_CH
            pltpu.make_async_copy(row_hbm.at[pl.ds(base, _SC_CH)], row_v[bu],
                                  sem_ri[bu]).wait()
            pltpu.make_async_copy(col_hbm.at[pl.ds(base, _SC_CH)], col_v[bu],
                                  sem_ci[bu]).wait()
            pltpu.async_copy(x_hbm.at[row_v[bu]], abuf[bu], sem_a[bu])
            pltpu.async_copy(x_hbm.at[col_v[bu]], bbuf[bu], sem_b[bu])

        def c_stage(ci, bu):
            base = wid * perw + ci * _SC_CH
            pltpu.make_async_copy(x_hbm.at[row_v[bu]], abuf[bu],
                                  sem_a[bu]).wait()
            pltpu.make_async_copy(x_hbm.at[col_v[bu]], bbuf[bu],
                                  sem_b[bu]).wait()
            ab, bb = abuf[bu], bbuf[bu]

            def rbody(r2, cr):
                for rr in range(2):
                    r = r2 * 2 + rr
                    for u in range(_H // 16):
                        sl = pl.ds(u * 16, 16)
                        ab[r, sl] = (ab[r, sl] + bb[r, sl]) * 0.5
                return cr

            lax.fori_loop(0, _SC_CH // 2, rbody, 0)
            pltpu.sync_copy(ab, out_hbm.at[pl.ds(base, _SC_CH)])

        a_stage(0, 0)
        a_stage(1, 1)
        b_stage(0, 0)

        def gbody(gi, cr):
            b_stage(2 * gi + 1, 1)
            c_stage(2 * gi, 0)

            @pl.when(gi < nch // 2 - 1)
            def _n0():
                a_stage(2 * gi + 2, 0)
                b_stage(2 * gi + 2, 0)

            c_stage(2 * gi + 1, 1)

            @pl.when(gi < nch // 2 - 1)
            def _n1():
                a_stage(2 * gi + 3, 1)

            return cr

        lax.fori_loop(0, nch // 2, gbody, 0)

    return k(x, row_idx, col_idx)


def kernel(node_features, edge_features, edge_index, global_features,
           timestep, params):
    del global_features
    p = params
    f32 = jnp.float32

    x0 = jnp.pad(node_features, ((0, _NP - _N), (0, 0)))
    ef = jnp.pad(edge_features, ((0, _EP - _E), (0, 0)))
    row = edge_index[0]
    col = edge_index[1]
    rowp = jnp.concatenate([row, jnp.zeros((_EP - _E,), jnp.int32)])
    colp = jnp.concatenate(
        [col, jnp.full((_EP - _E,), _NP - 1, jnp.int32)])

    t = timestep.astype(f32)[:, None]
    h1 = t @ p["time_l1"]["w"].T + p["time_l1"]["b"]
    h1 = h1 * jax.nn.sigmoid(h1)
    t_emb = h1 @ p["time_l2"]["w"].T + p["time_l2"]["b"]

    node_b = p["node_proj"]["b"][None, :] + t_emb
    x = _mm(x0, p["node_proj"]["w"].T, node_b, bm=512)

    for lp in p["layers"]:
        wm = lp["mlp_lin"]["w"]
        wr_, wc_, we_ = wm[:, :_H], wm[:, _H:2 * _H], wm[:, 2 * _H:]
        big_w = jnp.concatenate(
            [lp["q"]["w"].T, lp["k"]["w"].T, lp["v"]["w"].T, wr_.T, wc_.T],
            axis=1)
        big_b = jnp.concatenate(
            [lp["q"]["b"], lp["k"]["b"], lp["v"]["b"],
             jnp.zeros((2 * _H,), f32)])[None, :]
        big = _mm(x, big_w, big_b, bm=512)
        q = big[:, :_H]
        kk = big[:, _H:2 * _H]
        vv = big[:, 2 * _H:3 * _H]
        a_rows = big[:, 3 * _H:4 * _H]
        b_rows = big[:, 4 * _H:]

        ao = _flash_attention(q, kk, vv, _N)

        cw = p["edge_proj"]["w"].T @ we_.T
        cb = (p["edge_proj"]["b"] @ we_.T + lp["mlp_lin"]["b"])[None, :]
        c_rows = _mm(ef, cw, cb, bm=1280)

        nm = _sc_edge_layer(a_rows, b_rows, c_rows, rowp, colp,
                            lp["mlp_ln"]["g"], lp["mlp_ln"]["b"],
                            _NP)

        wout = lp["out"]["w"]
        x = _combine(
            ao, nm[0], nm[1], x,
            lp["o"]["w"].T, lp["o"]["b"][None, :],
            wout[:, :_H].T, wout[:, _H:].T, lp["out"]["b"][None, :],
            lp["ln"]["g"][None, :], lp["ln"]["b"][None, :], bm=512)

    node_noise = _head(
        x, p["node_out1"]["w"].T, p["node_out1"]["b"][None, :],
        p["node_out_ln"]["g"][None, :], p["node_out_ln"]["b"][None, :],
        p["node_out2"]["w"].T, p["node_out2"]["b"][None, :], bm=512)[:_N]

    g2 = _sc_gather_mean(x, rowp, colp)
    edge_noise = _head(
        g2, p["edge_out1"]["w"].T, p["edge_out1"]["b"][None, :],
        p["edge_out_ln"]["g"][None, :], p["edge_out_ln"]["b"][None, :],
        p["edge_out2"]["w"].T, p["edge_out2"]["b"][None, :], bm=1280)[:_E]

    topo = _mean_topo(
        x, _N, p["topo1"]["w"].T, p["topo1"]["b"][None, :],
        p["topo2"]["w"].T, p["topo2"]["b"][None, :], bm=512)
    topology_logits = topo.reshape(_MAXN, _MAXN)

    return (node_noise, edge_noise, topology_logits, x[:_N])


# speedup vs baseline: 1.0608x; 1.0195x over previous
import functools
import math

import jax
import jax.numpy as jnp
from jax import lax
from jax.experimental import pallas as pl
from jax.experimental.pallas import tpu as pltpu
from jax.experimental.pallas import tpu_sc as plsc

_N = 10000
_E = 160000
_H = 128
_NP = 10240
_EP = 163840
_MAXN = 50
_EPS = 1e-5
_NEG = -1e30

_SC_NW = 32
_SC_CH = 128


def _mm(x, w, b, bm):
    m, k = x.shape
    d = w.shape[1]

    def body(x_ref, w_ref, b_ref, o_ref):
        o_ref[...] = (
            jnp.dot(x_ref[...], w_ref[...], preferred_element_type=jnp.float32)
            + b_ref[...]
        )

    return pl.pallas_call(
        body,
        grid=(m // bm,),
        in_specs=[
            pl.BlockSpec((bm, k), lambda i: (i, 0)),
            pl.BlockSpec((k, d), lambda i: (0, 0)),
            pl.BlockSpec((1, d), lambda i: (0, 0)),
        ],
        out_specs=pl.BlockSpec((bm, d), lambda i: (i, 0)),
        out_shape=jax.ShapeDtypeStruct((m, d), jnp.float32),
    )(x, w, b)


def _mm_multi(x, w, b, bm, splits):
    m, k = x.shape
    d = w.shape[1]

    def body(x_ref, w_ref, b_ref, *o_refs):
        acc = (jnp.dot(x_ref[...], w_ref[...],
                       preferred_element_type=jnp.float32) + b_ref[...])
        ofs = 0
        for o_ref, wd in zip(o_refs, splits):
            o_ref[...] = acc[:, ofs:ofs + wd]
            ofs += wd

    return pl.pallas_call(
        body,
        grid=(m // bm,),
        in_specs=[
            pl.BlockSpec((bm, k), lambda i: (i, 0)),
            pl.BlockSpec((k, d), lambda i: (0, 0)),
            pl.BlockSpec((1, d), lambda i: (0, 0)),
        ],
        out_specs=[pl.BlockSpec((bm, wd), lambda i: (i, 0)) for wd in splits],
        out_shape=[jax.ShapeDtypeStruct((m, wd), jnp.float32)
                   for wd in splits],
    )(x, w, b)


def _flash_attention(qkv, n_valid, bq=512, bk=512):
    m = qkv.shape[0]
    d = qkv.shape[1] // 3
    nj = m // bk
    scale = 1.0 / math.sqrt(d)

    def body(q_ref, k_ref, v_ref, o_ref, m_ref, l_ref, acc_ref):
        j = pl.program_id(1)

        @pl.when(j == 0)
        def _init():
            m_ref[...] = jnp.full_like(m_ref, _NEG)
            l_ref[...] = jnp.zeros_like(l_ref)
            acc_ref[...] = jnp.zeros_like(acc_ref)

        s = lax.dot_general(
            q_ref[...], k_ref[...], (((1,), (1,)), ((), ())),
            preferred_element_type=jnp.float32,
        ) * scale
        cols = j * bk + lax.broadcasted_iota(jnp.int32, (bq, bk), 1)
        s = jnp.where(cols < n_valid, s, _NEG)

        m_prev = m_ref[...]
        m_new = jnp.maximum(m_prev, jnp.max(s, axis=1, keepdims=True))
        alpha = jnp.exp(m_prev - m_new)
        p = jnp.exp(s - m_new[:, :1])
        l_ref[...] = l_ref[...] * alpha + jnp.sum(p, axis=1, keepdims=True)
        acc_ref[...] = acc_ref[...] * alpha + jnp.dot(
            p, v_ref[...], preferred_element_type=jnp.float32
        )
        m_ref[...] = m_new

        @pl.when(j == nj - 1)
        def _fin():
            o_ref[...] = acc_ref[...] / l_ref[...]

    return pl.pallas_call(
        body,
        grid=(m // bq, nj),
        in_specs=[
            pl.BlockSpec((bq, d), lambda i, j: (i, 0)),
            pl.BlockSpec((bk, d), lambda i, j: (j, 1)),
            pl.BlockSpec((bk, d), lambda i, j: (j, 2)),
        ],
        out_specs=pl.BlockSpec((bq, d), lambda i, j: (i, 0)),
        out_shape=jax.ShapeDtypeStruct((m, d), jnp.float32),
        scratch_shapes=[
            pltpu.VMEM((bq, d), jnp.float32),
            pltpu.VMEM((bq, d), jnp.float32),
            pltpu.VMEM((bq, d), jnp.float32),
        ],
        compiler_params=pltpu.CompilerParams(
            dimension_semantics=("parallel", "arbitrary")
        ),
    )(qkv, qkv, qkv)


def _ln_val(h, g, b):
    mu = jnp.mean(h, axis=-1, keepdims=True)
    c = h - mu
    var = jnp.mean(c * c, axis=-1, keepdims=True)
    return c * lax.rsqrt(var + _EPS) * g + b


def _combine(ao, nm, x, wo_t, bo, wl_t, wr_t, bout, g, bln, bm):
    m, d = x.shape

    def body(ao_ref, n0_ref, n1_ref, x_ref, wo_ref, wl_ref, wr_ref,
             bo_ref, bout_ref, g_ref, b_ref, o_ref):
        xa = jnp.dot(ao_ref[...], wo_ref[...],
                     preferred_element_type=jnp.float32) + bo_ref[...]
        h = (
            jnp.dot(xa, wl_ref[...], preferred_element_type=jnp.float32)
            + jnp.dot(n0_ref[0] + n1_ref[0], wr_ref[...],
                      preferred_element_type=jnp.float32)
            + bout_ref[...]
            + x_ref[...]
        )
        o_ref[...] = _ln_val(h, g_ref[...], b_ref[...])

    mat = pl.BlockSpec((bm, d), lambda i: (i, 0))
    nspec0 = pl.BlockSpec((1, bm, d), lambda i: (0, i, 0))
    nspec1 = pl.BlockSpec((1, bm, d), lambda i: (1, i, 0))
    wspec = pl.BlockSpec((d, d), lambda i: (0, 0))
    vec = pl.BlockSpec((1, d), lambda i: (0, 0))
    return pl.pallas_call(
        body,
        grid=(m // bm,),
        in_specs=[mat, nspec0, nspec1, mat, wspec, wspec, wspec,
                  vec, vec, vec, vec],
        out_specs=mat,
        out_shape=jax.ShapeDtypeStruct((m, d), jnp.float32),
    )(ao, nm, nm, x, wo_t, wl_t, wr_t, bo, bout, g, bln)


def _head(x, w1_t, b1, g, bln, w2_t, b2, bm):
    m, d = x.shape
    d1 = w1_t.shape[1]
    d2 = w2_t.shape[1]

    def body(x_ref, w1_ref, b1_ref, g_ref, bln_ref, w2_ref, b2_ref, o_ref):
        h = jnp.dot(x_ref[...], w1_ref[...],
                    preferred_element_type=jnp.float32) + b1_ref[...]
        h = _ln_val(h, g_ref[...], bln_ref[...])
        h = h * (1.0 / (1.0 + jnp.exp(-h)))
        o_ref[...] = jnp.dot(h, w2_ref[...],
                             preferred_element_type=jnp.float32) + b2_ref[...]

    return pl.pallas_call(
        body,
        grid=(m // bm,),
        in_specs=[
            pl.BlockSpec((bm, d), lambda i: (i, 0)),
            pl.BlockSpec((d, d1), lambda i: (0, 0)),
            pl.BlockSpec((1, d1), lambda i: (0, 0)),
            pl.BlockSpec((1, d1), lambda i: (0, 0)),
            pl.BlockSpec((1, d1), lambda i: (0, 0)),
            pl.BlockSpec((d1, d2), lambda i: (0, 0)),
            pl.BlockSpec((1, d2), lambda i: (0, 0)),
        ],
        out_specs=pl.BlockSpec((bm, d2), lambda i: (i, 0)),
        out_shape=jax.ShapeDtypeStruct((m, d2), jnp.float32),
    )(x, w1_t, b1, g, bln, w2_t, b2)


def _mean_topo(x, n_valid, w1_t, b1, w2_t, b2, bm):
    m, d = x.shape
    d1 = w1_t.shape[1]
    d2 = w2_t.shape[1]
    nb = m // bm

    def body(x_ref, w1_ref, b1_ref, w2_ref, b2_ref, o_ref, acc_ref):
        i = pl.program_id(0)

        @pl.when(i == 0)
        def _init():
            acc_ref[...] = jnp.zeros_like(acc_ref)

        rows = i * bm + lax.broadcasted_iota(jnp.int32, (bm, d), 0)
        xm = jnp.where(rows < n_valid, x_ref[...], 0.0)
        acc_ref[...] = acc_ref[...] + jnp.sum(xm, axis=0, keepdims=True)

        @pl.when(i == nb - 1)
        def _fin():
            gr = acc_ref[...] * (1.0 / n_valid)
            h = jnp.maximum(
                jnp.dot(gr, w1_ref[...], preferred_element_type=jnp.float32)
                + b1_ref[...], 0.0)
            o_ref[...] = jnp.dot(h, w2_ref[...],
                                 preferred_element_type=jnp.float32) + b2_ref[...]

    return pl.pallas_call(
        body,
        grid=(nb,),
        in_specs=[
            pl.BlockSpec((bm, d), lambda i: (i, 0)),
            pl.BlockSpec((d, d1), lambda i: (0, 0)),
            pl.BlockSpec((1, d1), lambda i: (0, 0)),
            pl.BlockSpec((d1, d2), lambda i: (0, 0)),
            pl.BlockSpec((1, d2), lambda i: (0, 0)),
        ],
        out_specs=pl.BlockSpec((1, d2), lambda i: (0, 0)),
        out_shape=jax.ShapeDtypeStruct((1, d2), jnp.float32),
        scratch_shapes=[pltpu.VMEM((1, d), jnp.float32)],
        compiler_params=pltpu.CompilerParams(
            dimension_semantics=("arbitrary",)
        ),
    )(x, w1_t, b1, w2_t, b2)


@functools.cache
def _sc_mesh():
    return plsc.VectorSubcoreMesh(core_axis_name="c", subcore_axis_name="s")


def _sc_edge_layer(a, b, c, row_idx, col_idx, g, bln, n_rows):
    ep = c.shape[0]
    perw = ep // _SC_NW
    ch = 64
    nch = perw // ch
    rpt = n_rows // 16
    idx_t = pltpu.VMEM((ch,), jnp.int32)
    buf_t = pltpu.VMEM((ch, _H), jnp.float32)
    sem_t = pltpu.SemaphoreType.DMA

    @functools.partial(
        pl.kernel,
        mesh=_sc_mesh(),
        out_type=jax.ShapeDtypeStruct((2, n_rows, _H), jnp.float32),
        scratch_types=[
            (idx_t, idx_t), (idx_t, idx_t),
            (buf_t, buf_t), (buf_t, buf_t), buf_t,
            pltpu.VMEM((_H,), jnp.float32),
            pltpu.VMEM((_H,), jnp.float32),
            pltpu.VMEM_SHARED((n_rows, _H), jnp.float32),
            (sem_t, sem_t), (sem_t, sem_t), sem_t,
            (sem_t, sem_t), (sem_t, sem_t),
        ],
    )
    def k(a_hbm, b_hbm, c_hbm, row_hbm, col_hbm, g_hbm, bln_hbm, out_hbm,
          row_v, col_v, abuf, bbuf, cbuf, gbuf, blnbuf, acc,
          sem_a, sem_b, sem_c, sem_ri, sem_ci):
        cid = lax.axis_index("c")
        sid = lax.axis_index("s")
        wid = sid * 2 + cid

        pltpu.sync_copy(g_hbm, gbuf)
        pltpu.sync_copy(bln_hbm, blnbuf)

        def zb(r, cr):
            for u in range(_H // 16):
                cbuf[r, pl.ds(u * 16, 16)] = jnp.zeros((16,), jnp.float32)
            return cr

        lax.fori_loop(0, ch, zb, 0)
        for t in range(rpt // ch):
            pltpu.sync_copy(cbuf, acc.at[pl.ds(sid * rpt + t * ch, ch)])
        plsc.subcore_barrier()

        def a_stage(ci, bu):
            base = wid * perw + ci * ch
            pltpu.async_copy(row_hbm.at[pl.ds(base, ch)], row_v[bu],
                             sem_ri[bu])
            pltpu.async_copy(col_hbm.at[pl.ds(base, ch)], col_v[bu],
                             sem_ci[bu])

        def b_stage(ci, bu):
            base = wid * perw + ci * ch
            pltpu.make_async_copy(row_hbm.at[pl.ds(base, ch)], row_v[bu],
                                  sem_ri[bu]).wait()
            pltpu.make_async_copy(col_hbm.at[pl.ds(base, ch)], col_v[bu],
                                  sem_ci[bu]).wait()
            pltpu.async_copy(a_hbm.at[row_v[bu]], abuf[bu], sem_a[bu])
            pltpu.async_copy(b_hbm.at[col_v[bu]], bbuf[bu], sem_b[bu])

        def c_stage(ci, bu):
            base = wid * perw + ci * ch
            cpc = pltpu.async_copy(c_hbm.at[pl.ds(base, ch)], cbuf, sem_c)
            pltpu.make_async_copy(a_hbm.at[row_v[bu]], abuf[bu],
                                  sem_a[bu]).wait()
            pltpu.make_async_copy(b_hbm.at[col_v[bu]], bbuf[bu],
                                  sem_b[bu]).wait()
            cpc.wait()
            ab, bb, cb = abuf[bu], bbuf[bu], cbuf

            def one_row(r):
                t = [ab[r, pl.ds(u * 16, 16)] + bb[r, pl.ds(u * 16, 16)]
                     + cb[r, pl.ds(u * 16, 16)] for u in range(_H // 16)]
                s = t[0]
                sq = t[0] * t[0]
                for u in range(1, _H // 16):
                    s = s + t[u]
                    sq = sq + t[u] * t[u]
                lane = lax.iota(jnp.int32, 16)
                dnum = lax.GatherDimensionNumbers(
                    offset_dims=(), collapsed_slice_dims=(0,),
                    start_index_map=(0,))

                def shuf(v, perm):
                    return lax.gather(
                        v, perm[:, None], dnum, slice_sizes=(1,),
                        mode=lax.GatherScatterMode.PROMISE_IN_BOUNDS)

                for kk in (1, 2, 4, 8):
                    perm = lane ^ kk
                    s = s + shuf(s, perm)
                    sq = sq + shuf(sq, perm)
                meanv = s * (1.0 / _H)
                var = sq * (1.0 / _H) - meanv * meanv
                xv = var + _EPS
                xi = lax.bitcast_convert_type(xv, jnp.int32)
                yi = 0x5F3759DF - lax.shift_right_arithmetic(xi, 1)
                y = lax.bitcast_convert_type(yi, jnp.float32)
                y = y * (1.5 - 0.5 * xv * y * y)
                y = y * (1.5 - 0.5 * xv * y * y)
                y = y * (1.5 - 0.5 * xv * y * y)
                for u in range(_H // 16):
                    sl = pl.ds(u * 16, 16)
                    o = (t[u] - meanv) * y * gbuf[sl] + blnbuf[sl]
                    cb[r, sl] = jnp.maximum(o, 0.0)

            def rbody(r2, cr):
                one_row(r2 * 2)
                one_row(r2 * 2 + 1)
                return cr

            lax.fori_loop(0, ch // 2, rbody, 0)
            pltpu.sync_copy(cb, acc.at[col_v[bu]], add=True)

        a_stage(0, 0)
        a_stage(1, 1)
        b_stage(0, 0)

        def gbody(gi, cr):
            b_stage(2 * gi + 1, 1)
            c_stage(2 * gi, 0)

            @pl.when(gi < nch // 2 - 1)
            def _n0():
                a_stage(2 * gi + 2, 0)
                b_stage(2 * gi + 2, 0)

            c_stage(2 * gi + 1, 1)

            @pl.when(gi < nch // 2 - 1)
            def _n1():
                a_stage(2 * gi + 3, 1)

            return cr

        lax.fori_loop(0, nch // 2, gbody, 0)
        plsc.subcore_barrier()
        pltpu.sync_copy(acc.at[pl.ds(sid * rpt, rpt)],
                        out_hbm.at[cid, pl.ds(sid * rpt, rpt)])

    return k(a, b, c, row_idx, col_idx, g, bln)


def _sc_gather_mean(x, row_idx, col_idx):
    ep = row_idx.shape[0]
    perw = ep // _SC_NW
    nch = perw // _SC_CH

    idx_t = pltpu.VMEM((_SC_CH,), jnp.int32)
    buf_t = pltpu.VMEM((_SC_CH, _H), jnp.float32)
    sem_t = pltpu.SemaphoreType.DMA

    @functools.partial(
        pl.kernel,
        mesh=_sc_mesh(),
        out_type=jax.ShapeDtypeStruct((ep, _H), jnp.float32),
        scratch_types=[
            (idx_t, idx_t), (idx_t, idx_t),
            (buf_t, buf_t), (buf_t, buf_t),
            (sem_t, sem_t), (sem_t, sem_t),
            (sem_t, sem_t), (sem_t, sem_t),
        ],
    )
    def k(x_hbm, row_hbm, col_hbm, out_hbm,
          row_v, col_v, abuf, bbuf, sem_a, sem_b, sem_ri, sem_ci):
        wid = lax.axis_index("s") * 2 + lax.axis_index("c")

        def a_stage(ci, bu):
            base = wid * perw + ci * _SC_CH
            pltpu.async_copy(row_hbm.at[pl.ds(base, _SC_CH)], row_v[bu],
                             sem_ri[bu])
            pltpu.async_copy(col_hbm.at[pl.ds(base, _SC_CH)], col_v[bu],
                             sem_ci[bu])

        def b_stage(ci, bu):
            base = wid * perw + ci * _SC_CH
            pltpu.make_async_copy(row_hbm.at[pl.ds(base, _SC_CH)], row_v[bu],
                                  sem_ri[bu]).wait()
            pltpu.make_async_copy(col_hbm.at[pl.ds(base, _SC_CH)], col_v[bu],
                                  sem_ci[bu]).wait()
            pltpu.async_copy(x_hbm.at[row_v[bu]], abuf[bu], sem_a[bu])
            pltpu.async_copy(x_hbm.at[col_v[bu]], bbuf[bu], sem_b[bu])

        def c_stage(ci, bu):
            base = wid * perw + ci * _SC_CH
            pltpu.make_async_copy(x_hbm.at[row_v[bu]], abuf[bu],
                                  sem_a[bu]).wait()
            pltpu.make_async_copy(x_hbm.at[col_v[bu]], bbuf[bu],
                                  sem_b[bu]).wait()
            ab, bb = abuf[bu], bbuf[bu]

            def rbody(r2, cr):
                for rr in range(2):
                    r = r2 * 2 + rr
                    for u in range(_H // 16):
                        sl = pl.ds(u * 16, 16)
                        ab[r, sl] = (ab[r, sl] + bb[r, sl]) * 0.5
                return cr

            lax.fori_loop(0, _SC_CH // 2, rbody, 0)
            pltpu.sync_copy(ab, out_hbm.at[pl.ds(base, _SC_CH)])

        a_stage(0, 0)
        a_stage(1, 1)
        b_stage(0, 0)

        def gbody(gi, cr):
            b_stage(2 * gi + 1, 1)
            c_stage(2 * gi, 0)

            @pl.when(gi < nch // 2 - 1)
            def _n0():
                a_stage(2 * gi + 2, 0)
                b_stage(2 * gi + 2, 0)

            c_stage(2 * gi + 1, 1)

            @pl.when(gi < nch // 2 - 1)
            def _n1():
                a_stage(2 * gi + 3, 1)

            return cr

        lax.fori_loop(0, nch // 2, gbody, 0)

    return k(x, row_idx, col_idx)


def kernel(node_features, edge_features, edge_index, global_features,
           timestep, params):
    del global_features
    p = params
    f32 = jnp.float32

    x0 = jnp.pad(node_features, ((0, _NP - _N), (0, 0)))
    ef = jnp.pad(edge_features, ((0, _EP - _E), (0, 0)))
    row = edge_index[0]
    col = edge_index[1]
    rowp = jnp.concatenate([row, jnp.zeros((_EP - _E,), jnp.int32)])
    colp = jnp.concatenate(
        [col, jnp.full((_EP - _E,), _NP - 1, jnp.int32)])

    t = timestep.astype(f32)[:, None]
    h1 = t @ p["time_l1"]["w"].T + p["time_l1"]["b"]
    h1 = h1 * jax.nn.sigmoid(h1)
    t_emb = h1 @ p["time_l2"]["w"].T + p["time_l2"]["b"]

    node_b = p["node_proj"]["b"][None, :] + t_emb
    x = _mm(x0, p["node_proj"]["w"].T, node_b, bm=512)

    for lp in p["layers"]:
        wm = lp["mlp_lin"]["w"]
        wr_, wc_, we_ = wm[:, :_H], wm[:, _H:2 * _H], wm[:, 2 * _H:]
        big_w = jnp.concatenate(
            [lp["q"]["w"].T, lp["k"]["w"].T, lp["v"]["w"].T, wr_.T, wc_.T],
            axis=1)
        big_b = jnp.concatenate(
            [lp["q"]["b"], lp["k"]["b"], lp["v"]["b"],
             jnp.zeros((2 * _H,), f32)])[None, :]
        qkv, a_rows, b_rows = _mm_multi(x, big_w, big_b, 512,
                                        (3 * _H, _H, _H))
        ao = _flash_attention(qkv, _N)

        cw = p["edge_proj"]["w"].T @ we_.T
        cb = (p["edge_proj"]["b"] @ we_.T + lp["mlp_lin"]["b"])[None, :]
        c_rows = _mm(ef, cw, cb, bm=1280)

        nm = _sc_edge_layer(a_rows, b_rows, c_rows, rowp, colp,
                            lp["mlp_ln"]["g"], lp["mlp_ln"]["b"],
                            _NP)

        wout = lp["out"]["w"]
        x = _combine(
            ao, nm, x,
            lp["o"]["w"].T, lp["o"]["b"][None, :],
            wout[:, :_H].T, wout[:, _H:].T, lp["out"]["b"][None, :],
            lp["ln"]["g"][None, :], lp["ln"]["b"][None, :], bm=512)

    node_noise = _head(
        x, p["node_out1"]["w"].T, p["node_out1"]["b"][None, :],
        p["node_out_ln"]["g"][None, :], p["node_out_ln"]["b"][None, :],
        p["node_out2"]["w"].T, p["node_out2"]["b"][None, :], bm=512)[:_N]

    g2 = _sc_gather_mean(x, rowp, colp)
    edge_noise = _head(
        g2, p["edge_out1"]["w"].T, p["edge_out1"]["b"][None, :],
        p["edge_out_ln"]["g"][None, :], p["edge_out_ln"]["b"][None, :],
        p["edge_out2"]["w"].T, p["edge_out2"]["b"][None, :], bm=1280)[:_E]

    topo = _mean_topo(
        x, _N, p["topo1"]["w"].T, p["topo1"]["b"][None, :],
        p["topo2"]["w"].T, p["topo2"]["b"][None, :], bm=512)
    topology_logits = topo.reshape(_MAXN, _MAXN)

    return (node_noise, edge_noise, topology_logits, x[:_N])
